# baseline (speedup 1.0000x reference)
import jax
import jax.numpy as jnp
from jax import lax
from jax.experimental import pallas as pl
from jax.experimental.pallas import tpu as pltpu
from jax.experimental.pallas import tpu_sc as plsc

N = 10000
D_IN = 128
D_H = 256
G = 512
BLK = 1000
GRID = N // BLK

NC, NS = 2, 16
NW = NC * NS
GPT = G // NW
CH = 128
BUF = CH + 16
MAXC = (N + CH - 1) // CH + 1
NK = D_H // 16


def _mm_body(x_ref, W0_ref, b0_ref, h_ref):
    h = jnp.dot(x_ref[...], W0_ref[...], preferred_element_type=jnp.float32)
    h = h + b0_ref[...]
    h_ref[...] = jnp.where(h > 0, h, jnp.exp(jnp.minimum(h, 0.0)) - 1.0)


def _stage1(x, W0, b0):
    return pl.pallas_call(
        _mm_body,
        grid=(GRID,),
        in_specs=[
            pl.BlockSpec((BLK, D_IN), lambda i: (i, 0)),
            pl.BlockSpec((D_IN, D_H), lambda i: (0, 0)),
            pl.BlockSpec((1, D_H), lambda i: (0, 0)),
        ],
        out_specs=pl.BlockSpec((BLK, D_H), lambda i: (i, 0)),
        out_shape=jax.ShapeDtypeStruct((N, D_H), jnp.float32),
    )(x, W0, b0.reshape(1, D_H))


def _gat(ref, i):
    return plsc.load_gather(ref, [jnp.full((16,), i, jnp.int32)])[0]


def _sc_body(h_hbm, starts_hbm, glo_hbm, ghi_hbm, out_hbm,
             buf0, buf1, ostage, st_v, glo_v, ghi_v, sem0, sem1):
    c = lax.axis_index("c")
    s = lax.axis_index("s")
    wid = c * NS + s
    g0 = wid * GPT

    pltpu.sync_copy(starts_hbm.at[pl.ds(g0, 24)], st_v)
    pltpu.sync_copy(glo_hbm.at[wid], glo_v)
    pltpu.sync_copy(ghi_hbm.at[wid], ghi_v)
    sv0 = st_v[pl.ds(0, 16)]
    sv1 = st_v[pl.ds(8, 16)]
    row_s = sv0[0]
    row_e = sv1[8]

    zv = jnp.zeros((16,), jnp.float32)
    for r in range(GPT):
        for k in range(NK):
            ostage[r, pl.ds(k * 16, 16)] = zv

    nchunks = lax.div(row_e - row_s + (CH - 1), CH)

    def chunk_start(ci):
        base = row_s + ci * CH
        start = jnp.minimum(base, N - BUF)
        return base, (start // 8) * 8

    del buf1, sem0, sem1

    def process(ci, buf):
        base, start = chunk_start(ci)
        cnt = jnp.minimum(row_e - base, CH)
        chunk_end = base + cnt
        gl_lo = _gat(glo_v, ci)
        gl_hi = _gat(ghi_v, ci)

        def graph_body(g, _):
            s_g = _gat(st_v, g)
            e_g = _gat(st_v, g + 1)
            lo = jnp.maximum(s_g, base) - start
            hi = jnp.minimum(e_g, chunk_end) - start

            def row_body(slot, acc):
                return tuple(acc[k] + buf[slot, pl.ds(k * 16, 16)]
                             for k in range(NK))

            acc = lax.fori_loop(lo, hi, row_body,
                                tuple(zv for _ in range(NK)))
            for k in range(NK):
                plsc.addupdate(ostage.at[g, pl.ds(k * 16, 16)], acc[k])
            return 0

        lax.fori_loop(gl_lo, gl_hi, graph_body, 0)

    def chunk_loop(ci, _):
        _, start = chunk_start(ci)
        pltpu.sync_copy(h_hbm.at[pl.ds(start, BUF)], buf0)
        process(ci, buf0)
        return 0

    lax.fori_loop(0, nchunks, chunk_loop, 0)

    pltpu.sync_copy(ostage, out_hbm.at[pl.ds(g0, GPT)])


def _sc_segment_sum(h, starts, glo, ghi):
    mesh = plsc.VectorSubcoreMesh(core_axis_name="c", subcore_axis_name="s")
    f = pl.kernel(
        _sc_body,
        out_type=jax.ShapeDtypeStruct((G, D_H), jnp.float32),
        mesh=mesh,
        compiler_params=pltpu.CompilerParams(use_tc_tiling_on_sc=False,
                                             needs_layout_passes=False),
        scratch_types=[
            pltpu.VMEM((BUF, D_H), jnp.float32),
            pltpu.VMEM((BUF, D_H), jnp.float32),
            pltpu.VMEM((GPT, D_H), jnp.float32),
            pltpu.VMEM((24,), jnp.int32),
            pltpu.VMEM((MAXC,), jnp.int32),
            pltpu.VMEM((MAXC,), jnp.int32),
            pltpu.SemaphoreType.DMA,
            pltpu.SemaphoreType.DMA,
        ],
    )
    return f(h, starts, glo, ghi)


def _head_body(p_ref, W1_ref, b1_ref, w2_ref, b2_ref, out_ref):
    pooled = p_ref[...]
    h2 = jnp.dot(pooled, W1_ref[...], preferred_element_type=jnp.float32)
    h2 = jnp.maximum(h2 + b1_ref[...], 0.0)
    logit = jnp.sum(h2 * w2_ref[...], axis=1, keepdims=True) + b2_ref[...]
    out_ref[...] = 1.0 / (1.0 + jnp.exp(-logit))


def _head(pooled, W1, b1, W2, b2):
    return pl.pallas_call(
        _head_body,
        out_shape=jax.ShapeDtypeStruct((G, 1), jnp.float32),
    )(pooled, W1, b1.reshape(1, D_H), W2.reshape(1, D_H), b2.reshape(1, 1))


def _build_maps(batch):
    starts512 = jnp.searchsorted(
        batch, jnp.arange(G + 1, dtype=jnp.int32)).astype(jnp.int32)
    starts = jnp.concatenate([starts512, jnp.full((7,), N, jnp.int32)])

    t = jnp.arange(NW, dtype=jnp.int32)
    rs = starts512[t * GPT]
    re = starts512[(t + 1) * GPT]
    ci = jnp.arange(MAXC, dtype=jnp.int32)
    base = rs[:, None] + ci[None, :] * CH
    lr = jnp.minimum(base + CH, re[:, None]) - 1
    valid = lr >= base
    g_of_base = jnp.searchsorted(starts512, base, side="right").astype(
        jnp.int32) - 1
    g_of_lr = jnp.searchsorted(starts512, lr, side="right").astype(
        jnp.int32) - 1
    glo = jnp.clip(g_of_base - (t * GPT)[:, None], 0, GPT)
    ghi = jnp.clip(g_of_lr + 1 - (t * GPT)[:, None], 0, GPT)
    glo = jnp.where(valid, glo, 0)
    ghi = jnp.where(valid, ghi, 0)
    return starts, glo, ghi


def kernel(x, edge_index, batch, W0, b0, W1, b1, W2, b2):
    del edge_index
    h = _stage1(x, W0, b0)
    starts, glo, ghi = _build_maps(batch)
    pooled = _sc_segment_sum(h, starts, glo, ghi)
    out = _head(pooled, W1, b1, W2, b2)
    return out.reshape(G)


# speedup vs baseline: 2.1307x; 2.1307x over previous
import jax
import jax.numpy as jnp
from jax import lax
from jax.experimental import pallas as pl
from jax.experimental.pallas import tpu as pltpu
from jax.experimental.pallas import tpu_sc as plsc

N = 10000
D_IN = 128
D_H = 256
G = 512
BLK = 1000
GRID = N // BLK

NC, NS = 2, 16
NW = NC * NS
GPT = G // NW
CH = 128
BUF = CH + 16
MAXC = (N + CH - 1) // CH + 1
NK = D_H // 16


def _mm_body(x_ref, W0_ref, b0_ref, h_ref):
    h = jnp.dot(x_ref[...], W0_ref[...], preferred_element_type=jnp.float32)
    h = h + b0_ref[...]
    h_ref[...] = jnp.where(h > 0, h, jnp.exp(jnp.minimum(h, 0.0)) - 1.0)


def _stage1(x, W0, b0):
    return pl.pallas_call(
        _mm_body,
        grid=(GRID,),
        in_specs=[
            pl.BlockSpec((BLK, D_IN), lambda i: (i, 0)),
            pl.BlockSpec((D_IN, D_H), lambda i: (0, 0)),
            pl.BlockSpec((1, D_H), lambda i: (0, 0)),
        ],
        out_specs=pl.BlockSpec((BLK, D_H), lambda i: (i, 0)),
        out_shape=jax.ShapeDtypeStruct((N, D_H), jnp.float32),
    )(x, W0, b0.reshape(1, D_H))


def _gat(ref, i):
    return plsc.load_gather(ref, [jnp.full((16,), i, jnp.int32)])[0]


def _sc_body(h_hbm, starts_hbm, glo_hbm, ghi_hbm, out_hbm,
             buf0, buf1, ostage, st_v, glo_v, ghi_v, sem0, sem1):
    c = lax.axis_index("c")
    s = lax.axis_index("s")
    wid = c * NS + s
    g0 = wid * GPT

    pltpu.sync_copy(starts_hbm.at[pl.ds(g0, 24)], st_v)
    pltpu.sync_copy(glo_hbm.at[wid], glo_v)
    pltpu.sync_copy(ghi_hbm.at[wid], ghi_v)
    sv0 = st_v[pl.ds(0, 16)]
    sv1 = st_v[pl.ds(8, 16)]
    row_s = sv0[0]
    row_e = sv1[8]

    zv = jnp.zeros((16,), jnp.float32)
    for r in range(GPT):
        for k in range(NK):
            ostage[r, pl.ds(k * 16, 16)] = zv

    nchunks = lax.div(row_e - row_s + (CH - 1), CH)

    def chunk_start(ci):
        base = row_s + ci * CH
        start = jnp.minimum(base, N - BUF)
        return base, (start // 8) * 8

    del buf1, sem0, sem1

    def process(ci, buf):
        base, start = chunk_start(ci)
        cnt = jnp.minimum(row_e - base, CH)
        chunk_end = base + cnt
        gl_lo = _gat(glo_v, ci)
        gl_hi = _gat(ghi_v, ci)

        def graph_body(g, _):
            s_g = _gat(st_v, g)
            e_g = _gat(st_v, g + 1)
            lo = jnp.maximum(s_g, base) - start
            hi = jnp.minimum(e_g, chunk_end) - start

            def row_body(slot, acc):
                return tuple(acc[k] + buf[slot, pl.ds(k * 16, 16)]
                             for k in range(NK))

            acc = lax.fori_loop(lo, hi, row_body,
                                tuple(zv for _ in range(NK)))
            for k in range(NK):
                plsc.addupdate(ostage.at[g, pl.ds(k * 16, 16)], acc[k])
            return 0

        lax.fori_loop(gl_lo, gl_hi, graph_body, 0)

    def chunk_loop(ci, _):
        _, start = chunk_start(ci)
        pltpu.sync_copy(h_hbm.at[pl.ds(start, BUF)], buf0)
        process(ci, buf0)
        return 0

    lax.fori_loop(0, nchunks, chunk_loop, 0)

    pltpu.sync_copy(ostage, out_hbm.at[pl.ds(g0, GPT)])


def _sc_segment_sum(h, starts, glo, ghi):
    mesh = plsc.VectorSubcoreMesh(core_axis_name="c", subcore_axis_name="s")
    f = pl.kernel(
        _sc_body,
        out_type=jax.ShapeDtypeStruct((G, D_H), jnp.float32),
        mesh=mesh,
        compiler_params=pltpu.CompilerParams(use_tc_tiling_on_sc=False,
                                             needs_layout_passes=False),
        scratch_types=[
            pltpu.VMEM((BUF, D_H), jnp.float32),
            pltpu.VMEM((BUF, D_H), jnp.float32),
            pltpu.VMEM((GPT, D_H), jnp.float32),
            pltpu.VMEM((24,), jnp.int32),
            pltpu.VMEM((MAXC,), jnp.int32),
            pltpu.VMEM((MAXC,), jnp.int32),
            pltpu.SemaphoreType.DMA,
            pltpu.SemaphoreType.DMA,
        ],
    )
    return f(h, starts, glo, ghi)


def _head_body(p_ref, W1_ref, b1_ref, w2_ref, b2_ref, out_ref):
    pooled = p_ref[...]
    h2 = jnp.dot(pooled, W1_ref[...], preferred_element_type=jnp.float32)
    h2 = jnp.maximum(h2 + b1_ref[...], 0.0)
    logit = jnp.sum(h2 * w2_ref[...], axis=1, keepdims=True) + b2_ref[...]
    out_ref[...] = 1.0 / (1.0 + jnp.exp(-logit))


def _head(pooled, W1, b1, W2, b2):
    return pl.pallas_call(
        _head_body,
        out_shape=jax.ShapeDtypeStruct((G, 1), jnp.float32),
    )(pooled, W1, b1.reshape(1, D_H), W2.reshape(1, D_H), b2.reshape(1, 1))


def _build_maps(batch):
    starts512 = jnp.searchsorted(
        batch, jnp.arange(G + 1, dtype=jnp.int32)).astype(jnp.int32)
    starts = jnp.concatenate([starts512, jnp.full((7,), N, jnp.int32)])

    t = jnp.arange(NW, dtype=jnp.int32)
    rs = starts512[t * GPT]
    re = starts512[(t + 1) * GPT]
    ci = jnp.arange(MAXC, dtype=jnp.int32)
    base = rs[:, None] + ci[None, :] * CH
    lr = jnp.minimum(base + CH, re[:, None]) - 1
    valid = lr >= base
    g_of_base = batch[jnp.clip(base, 0, N - 1)].astype(jnp.int32)
    g_of_lr = batch[jnp.clip(lr, 0, N - 1)].astype(jnp.int32)
    glo = jnp.clip(g_of_base - (t * GPT)[:, None], 0, GPT)
    ghi = jnp.clip(g_of_lr + 1 - (t * GPT)[:, None], 0, GPT)
    glo = jnp.where(valid, glo, 0)
    ghi = jnp.where(valid, ghi, 0)
    return starts, glo, ghi


def kernel(x, edge_index, batch, W0, b0, W1, b1, W2, b2):
    del edge_index
    h = _stage1(x, W0, b0)
    starts, glo, ghi = _build_maps(batch)
    pooled = _sc_segment_sum(h, starts, glo, ghi)
    out = _head(pooled, W1, b1, W2, b2)
    return out.reshape(G)


# speedup vs baseline: 2.9854x; 1.4011x over previous
import jax
import jax.numpy as jnp
from jax import lax
from jax.experimental import pallas as pl
from jax.experimental.pallas import tpu as pltpu
from jax.experimental.pallas import tpu_sc as plsc

N = 10000
D_IN = 128
D_H = 256
G = 512
BLK = 1000
GRID = N // BLK

NC, NS = 2, 16
NW = NC * NS
GPT = G // NW
CH = 128
BUF = CH + 16
MAXC = (N + CH - 1) // CH + 1
NK = D_H // 16


def _mm_body(x_ref, W0_ref, b0_ref, h_ref):
    h = jnp.dot(x_ref[...], W0_ref[...], preferred_element_type=jnp.float32)
    h = h + b0_ref[...]
    h_ref[...] = jnp.where(h > 0, h, jnp.exp(jnp.minimum(h, 0.0)) - 1.0)


def _stage1(x, W0, b0):
    return pl.pallas_call(
        _mm_body,
        grid=(GRID,),
        in_specs=[
            pl.BlockSpec((BLK, D_IN), lambda i: (i, 0)),
            pl.BlockSpec((D_IN, D_H), lambda i: (0, 0)),
            pl.BlockSpec((1, D_H), lambda i: (0, 0)),
        ],
        out_specs=pl.BlockSpec((BLK, D_H), lambda i: (i, 0)),
        out_shape=jax.ShapeDtypeStruct((N, D_H), jnp.float32),
    )(x, W0, b0.reshape(1, D_H))


def _gat(ref, i):
    return plsc.load_gather(ref, [jnp.full((16,), i, jnp.int32)])[0]


def _sc_body(h_hbm, starts_hbm, out_hbm, buf0, buf1, ostage, st_v, sem0, sem1):
    c = lax.axis_index("c")
    s = lax.axis_index("s")
    wid = c * NS + s
    g0 = wid * GPT

    pltpu.sync_copy(starts_hbm.at[pl.ds(g0, 24)], st_v)
    sv0 = st_v[pl.ds(0, 16)]
    sv1 = st_v[pl.ds(8, 16)]
    row_s = sv0[0]
    row_e = sv1[8]

    zv = jnp.zeros((16,), jnp.float32)
    for r in range(GPT):
        for k in range(NK):
            ostage[r, pl.ds(k * 16, 16)] = zv

    nchunks = lax.div(row_e - row_s + (CH - 1), CH)

    def chunk_start(ci):
        base = row_s + ci * CH
        start = jnp.minimum(base, N - BUF)
        return base, (start // 8) * 8

    del buf1, sem0, sem1

    def process(ci, buf):
        base, start = chunk_start(ci)
        cnt = jnp.minimum(row_e - base, CH)
        chunk_end = base + cnt

        def graph_body(g, _):
            s_g = _gat(st_v, g)
            e_g = _gat(st_v, g + 1)

            @pl.when((s_g < chunk_end) & (e_g > base))
            def _acc():
                lo = jnp.maximum(s_g, base) - start
                hi = jnp.minimum(e_g, chunk_end) - start

                def row_body(slot, acc):
                    return tuple(acc[k] + buf[slot, pl.ds(k * 16, 16)]
                                 for k in range(NK))

                acc = lax.fori_loop(lo, hi, row_body,
                                    tuple(zv for _ in range(NK)))
                for k in range(NK):
                    plsc.addupdate(ostage.at[g, pl.ds(k * 16, 16)], acc[k])

            return 0

        lax.fori_loop(0, GPT, graph_body, 0)

    def chunk_loop(ci, _):
        _, start = chunk_start(ci)
        pltpu.sync_copy(h_hbm.at[pl.ds(start, BUF)], buf0)
        process(ci, buf0)
        return 0

    lax.fori_loop(0, nchunks, chunk_loop, 0)

    pltpu.sync_copy(ostage, out_hbm.at[pl.ds(g0, GPT)])


def _sc_segment_sum(h, starts):
    mesh = plsc.VectorSubcoreMesh(core_axis_name="c", subcore_axis_name="s")
    f = pl.kernel(
        _sc_body,
        out_type=jax.ShapeDtypeStruct((G, D_H), jnp.float32),
        mesh=mesh,
        compiler_params=pltpu.CompilerParams(use_tc_tiling_on_sc=False,
                                             needs_layout_passes=False),
        scratch_types=[
            pltpu.VMEM((BUF, D_H), jnp.float32),
            pltpu.VMEM((BUF, D_H), jnp.float32),
            pltpu.VMEM((GPT, D_H), jnp.float32),
            pltpu.VMEM((24,), jnp.int32),
            pltpu.SemaphoreType.DMA,
            pltpu.SemaphoreType.DMA,
        ],
    )
    return f(h, starts)


def _head_body(p_ref, W1_ref, b1_ref, w2_ref, b2_ref, out_ref):
    pooled = p_ref[...]
    h2 = jnp.dot(pooled, W1_ref[...], preferred_element_type=jnp.float32)
    h2 = jnp.maximum(h2 + b1_ref[...], 0.0)
    logit = jnp.sum(h2 * w2_ref[...], axis=1, keepdims=True) + b2_ref[...]
    out_ref[...] = 1.0 / (1.0 + jnp.exp(-logit))


def _head(pooled, W1, b1, W2, b2):
    return pl.pallas_call(
        _head_body,
        out_shape=jax.ShapeDtypeStruct((G, 1), jnp.float32),
    )(pooled, W1, b1.reshape(1, D_H), W2.reshape(1, D_H), b2.reshape(1, 1))


def _build_starts(batch):
    starts512 = jnp.searchsorted(
        batch, jnp.arange(G + 1, dtype=jnp.int32)).astype(jnp.int32)
    return jnp.concatenate([starts512, jnp.full((7,), N, jnp.int32)])


def kernel(x, edge_index, batch, W0, b0, W1, b1, W2, b2):
    del edge_index
    h = _stage1(x, W0, b0)
    starts = _build_starts(batch)
    pooled = _sc_segment_sum(h, starts)
    out = _head(pooled, W1, b1, W2, b2)
    return out.reshape(G)
